# baseline (speedup 1.0000x reference)
import functools

import jax
import jax.numpy as jnp
from jax import lax
from jax.experimental import pallas as pl
from jax.experimental.pallas import tpu as pltpu
from jax.experimental.pallas import tpu_sc as plsc

N = 10000
E = 320000
D = 128
NC = 2
NS = 16
NW = NC * NS
EPT = E // NW
K = 80
NCH = EPT // K
NP_ = 10240
RPT = NP_ // NS
ZR = 32

_mesh = plsc.VectorSubcoreMesh(
    core_axis_name="c", subcore_axis_name="s", num_cores=NC, num_subcores=NS)


def _deg_body(col_hbm, w_hbm, out_hbm, colv, wv, degl):
    c = lax.axis_index("c")
    s = lax.axis_index("s")
    wid = c * NS + s
    base = wid * EPT

    def zero(i, _):
        degl[pl.ds(i * 16, 16)] = jnp.zeros((16,), jnp.float32)
        return 0
    lax.fori_loop(0, NP_ // 16, zero, 0)

    pltpu.sync_copy(col_hbm.at[pl.ds(base, EPT)], colv)
    pltpu.sync_copy(w_hbm.at[pl.ds(base, EPT)], wv)

    def scat(i, _):
        c16 = colv[pl.ds(i * 16, 16)]
        w16 = wv[pl.ds(i * 16, 16)]
        plsc.addupdate_scatter(degl, [c16], w16)
        return 0
    lax.fori_loop(0, EPT // 16, scat, 0)

    pltpu.sync_copy(degl, out_hbm.at[wid, 0])


_sc_params = pltpu.CompilerParams(needs_layout_passes=False)

_deg_call = pl.kernel(
    _deg_body,
    out_type=jax.ShapeDtypeStruct((NW, 1, NP_), jnp.float32),
    mesh=_mesh,
    compiler_params=_sc_params,
    scratch_types=[
        pltpu.VMEM((EPT,), jnp.int32),
        pltpu.VMEM((EPT,), jnp.float32),
        pltpu.VMEM((NP_,), jnp.float32),
    ],
)


NB = 3
NI = 6


def _agg_body(g_hbm, e_hbm, z_hbm, out_hbm,
              ebuf, rin, accs, isems, gsems, ssems):
    c = lax.axis_index("c")
    s = lax.axis_index("s")
    wid = c * NS + s
    base_ch = wid * NCH

    def zloop(i, _):
        pltpu.sync_copy(z_hbm, accs.at[pl.ds(s * RPT + i * ZR, ZR)])
        return 0
    lax.fori_loop(0, RPT // ZR, zloop, 0)
    plsc.subcore_barrier()

    def start_idx(j, b8):
        pltpu.async_copy(e_hbm.at[base_ch + j], ebuf.at[pl.ds(3 * b8, 3)], isems[b8])

    def wait_idx(j, b8):
        pltpu.make_async_copy(e_hbm.at[base_ch + j], ebuf.at[pl.ds(3 * b8, 3)],
                              isems[b8]).wait()

    def start_gather(b4, b8):
        pltpu.async_copy(g_hbm.at[ebuf.at[3 * b8, 0]], rin.at[b4], gsems[b4])

    def wait_gather(b4, b8):
        pltpu.make_async_copy(g_hbm.at[ebuf.at[3 * b8, 0]], rin.at[b4],
                              gsems[b4]).wait()

    def start_scat(b4, b8):
        pltpu.async_copy(rin.at[b4], accs.at[ebuf.at[3 * b8 + 1, 0]], ssems[b4],
                         add=True)

    def wait_scat(b4, b8):
        pltpu.make_async_copy(rin.at[b4], accs.at[ebuf.at[3 * b8 + 1, 0]],
                              ssems[b4]).wait()

    def scale(b4, b8):
        def one(e4, _):
            for u in range(4):
                e = e4 * 4 + u
                wspl = plsc.bitcast(plsc.load_gather(
                    ebuf, [jnp.full((16,), 3 * b8 + 2, jnp.int32),
                           jnp.zeros((16,), jnp.int32),
                           jnp.full((16,), e, jnp.int32)]), jnp.float32)
                for q in range(8):
                    sl = pl.ds(q * 16, 16)
                    rin[b4, e, sl] = rin[b4, e, sl] * wspl
            return 0
        lax.fori_loop(0, K // 4, one, 0)

    def do_step(j, b4, b8):
        wait_gather(b4, b8)
        scale(b4, b8)
        start_scat(b4, b8)

        @pl.when(j >= 1)
        def _():
            wait_scat((b4 + 2) % 3, (b8 + 5) % 6)

        @pl.when(j + 2 < NCH)
        def _():
            wait_idx(j + 2, (b8 + 2) % 6)
            start_gather((b4 + 2) % 3, (b8 + 2) % 6)

        @pl.when(j + 4 < NCH)
        def _():
            start_idx(j + 4, (b8 + 4) % 6)

    for j0 in range(4):
        start_idx(j0, j0)
    for j0 in range(2):
        wait_idx(j0, j0)
        start_gather(j0, j0)

    def step(i6, _):
        for bb in range(6):
            do_step(i6 * 6 + bb, bb % 3, bb)
        return 0
    NT = NCH // 6
    lax.fori_loop(0, NT, step, 0)
    for t in range(NCH - NT * 6):
        do_step(jnp.int32(NT * 6 + t), t % 3, t)
    wait_scat((NCH - 1) % 3, (NCH - 1) % 6)

    plsc.subcore_barrier()

    def ocp(i, _):
        off = s * RPT + i * ZR
        pltpu.sync_copy(accs.at[pl.ds(off, ZR)], out_hbm.at[c, pl.ds(off, ZR)])
        return 0
    lax.fori_loop(0, RPT // ZR, ocp, 0)


_agg_call = pl.kernel(
    _agg_body,
    out_type=jax.ShapeDtypeStruct((NC, NP_, D), jnp.float32),
    mesh=_mesh,
    compiler_params=_sc_params,
    scratch_types=[
        pltpu.VMEM((3 * NI, 1, K), jnp.int32),
        pltpu.VMEM((NB, K, D), jnp.float32),
        pltpu.VMEM_SHARED((NP_, D), jnp.float32),
        [pltpu.SemaphoreType.DMA] * NI,
        [pltpu.SemaphoreType.DMA] * NB,
        [pltpu.SemaphoreType.DMA] * NB,
    ],
)

BM = 2048
_GRID = NP_ // BM


def _dis_of(degp_ref):
    i = pl.program_id(0)
    deg = jnp.sum(degp_ref[:, 0, pl.ds(i * BM, BM)], axis=0) + 1.0
    return jnp.where(deg > 0, lax.rsqrt(jnp.maximum(deg, 1e-12)), 0.0)


def _tc1_body(x_ref, w_ref, degp_ref, o_ref):
    dis = _dis_of(degp_ref)
    h = jnp.dot(x_ref[...], w_ref[...], preferred_element_type=jnp.float32)
    o_ref[...] = dis[:, None] * h


def _tc2_body(p_ref, g_ref, degp_ref, b_ref, w_ref, o_ref):
    dis = _dis_of(degp_ref)
    s = p_ref[0] + p_ref[1] + g_ref[...]
    out1 = dis[:, None] * s + b_ref[...]
    h = jnp.dot(out1, w_ref[...], preferred_element_type=jnp.float32)
    o_ref[...] = dis[:, None] * h


def _tc3_body(p_ref, g_ref, degp_ref, b_ref, w_ref, bfc_ref, o_ref):
    dis = _dis_of(degp_ref)
    s = p_ref[0] + p_ref[1] + g_ref[...]
    out2 = dis[:, None] * s + b_ref[...]
    y = jnp.dot(out2, w_ref[...], preferred_element_type=jnp.float32)
    y = y + bfc_ref[...]
    o_ref[...] = jnp.round(jnp.clip(y, 0.0, 10.0))


_tc1 = pl.pallas_call(
    _tc1_body,
    grid=(_GRID,),
    in_specs=[
        pl.BlockSpec((BM, D), lambda i: (i, 0)),
        pl.BlockSpec((D, D), lambda i: (0, 0)),
        pl.BlockSpec((NW, 1, NP_), lambda i: (0, 0, 0)),
    ],
    out_specs=pl.BlockSpec((BM, D), lambda i: (i, 0)),
    out_shape=jax.ShapeDtypeStruct((NP_, D), jnp.float32),
)

_tc2 = pl.pallas_call(
    _tc2_body,
    grid=(_GRID,),
    in_specs=[
        pl.BlockSpec((NC, BM, D), lambda i: (0, i, 0)),
        pl.BlockSpec((BM, D), lambda i: (i, 0)),
        pl.BlockSpec((NW, 1, NP_), lambda i: (0, 0, 0)),
        pl.BlockSpec((1, D), lambda i: (0, 0)),
        pl.BlockSpec((D, D), lambda i: (0, 0)),
    ],
    out_specs=pl.BlockSpec((BM, D), lambda i: (i, 0)),
    out_shape=jax.ShapeDtypeStruct((NP_, D), jnp.float32),
)

_tc3 = pl.pallas_call(
    _tc3_body,
    grid=(_GRID,),
    in_specs=[
        pl.BlockSpec((NC, BM, D), lambda i: (0, i, 0)),
        pl.BlockSpec((BM, D), lambda i: (i, 0)),
        pl.BlockSpec((NW, 1, NP_), lambda i: (0, 0, 0)),
        pl.BlockSpec((1, D), lambda i: (0, 0)),
        pl.BlockSpec((D, 1), lambda i: (0, 0)),
        pl.BlockSpec((1, 1), lambda i: (0, 0)),
    ],
    out_specs=pl.BlockSpec((BM, 1), lambda i: (i, 0)),
    out_shape=jax.ShapeDtypeStruct((NP_, 1), jnp.float32),
)


def kernel(x, edge_index, edge_weight, W1, b1, W2, b2, Wfc, bfc):
    row = edge_index[0]
    col = edge_index[1]
    x_p = jnp.pad(x, ((0, NP_ - N), (0, 0)))
    wbits = lax.bitcast_convert_type(edge_weight, jnp.int32)
    edata = jnp.stack([row.reshape(E // K, K), col.reshape(E // K, K),
                       wbits.reshape(E // K, K)], axis=1).reshape(
                           E // K, 3, 1, K)
    zrows = jnp.zeros((ZR, D), jnp.float32)
    degp = _deg_call(col, edge_weight)
    g1 = _tc1(x_p, W1, degp)
    p = _agg_call(g1, edata, zrows)
    g2 = _tc2(p, g1, degp, b1.reshape(1, D), W2)
    q = _agg_call(g2, edata, zrows)
    y = _tc3(q, g2, degp, b2.reshape(1, D), Wfc, bfc.reshape(1, 1))
    return y[:N]


# speedup vs baseline: 1.1050x; 1.1050x over previous
import functools

import jax
import jax.numpy as jnp
from jax import lax
from jax.experimental import pallas as pl
from jax.experimental.pallas import tpu as pltpu
from jax.experimental.pallas import tpu_sc as plsc

N = 10000
E = 320000
D = 128
NC = 2
NS = 16
NW = NC * NS
EPT = E // NW
K = 80
NCH = EPT // K
NP_ = 10240
RPT = NP_ // NS
ZR = 32

_mesh = plsc.VectorSubcoreMesh(
    core_axis_name="c", subcore_axis_name="s", num_cores=NC, num_subcores=NS)


def _deg_body(col_hbm, w_hbm, out_hbm, colv, wv, degl):
    c = lax.axis_index("c")
    s = lax.axis_index("s")
    wid = c * NS + s
    base = wid * EPT

    def zero(i, _):
        degl[pl.ds(i * 16, 16)] = jnp.zeros((16,), jnp.float32)
        return 0
    lax.fori_loop(0, NP_ // 16, zero, 0)

    pltpu.sync_copy(col_hbm.at[pl.ds(base, EPT)], colv)
    pltpu.sync_copy(w_hbm.at[pl.ds(base, EPT)], wv)

    def scat(i, _):
        c16 = colv[pl.ds(i * 16, 16)]
        w16 = wv[pl.ds(i * 16, 16)]
        plsc.addupdate_scatter(degl, [c16], w16)
        return 0
    lax.fori_loop(0, EPT // 16, scat, 0)

    pltpu.sync_copy(degl, out_hbm.at[wid, 0])


_sc_params = pltpu.CompilerParams(needs_layout_passes=False)

_deg_call = pl.kernel(
    _deg_body,
    out_type=jax.ShapeDtypeStruct((NW, 1, NP_), jnp.float32),
    mesh=_mesh,
    compiler_params=_sc_params,
    scratch_types=[
        pltpu.VMEM((EPT,), jnp.int32),
        pltpu.VMEM((EPT,), jnp.float32),
        pltpu.VMEM((NP_,), jnp.float32),
    ],
)


NB = 3
NI = 6


def _agg_body(g_hbm, row_hbm, col_hbm, w_hbm, z_hbm, out_hbm,
              rowc, colc, wc, rin, accs, isems, gsems, ssems):
    c = lax.axis_index("c")
    s = lax.axis_index("s")
    wid = c * NS + s
    base_ch = wid * NCH

    def zloop(i, _):
        pltpu.sync_copy(z_hbm, accs.at[pl.ds(s * RPT + i * ZR, ZR)])
        return 0
    lax.fori_loop(0, RPT // ZR, zloop, 0)
    plsc.subcore_barrier()

    def start_idx(j, b8):
        pltpu.async_copy(row_hbm.at[base_ch + j], rowc.at[b8], isems[b8])
        pltpu.async_copy(col_hbm.at[base_ch + j], colc.at[b8], isems[b8])
        pltpu.async_copy(w_hbm.at[base_ch + j], wc.at[b8], isems[b8])

    def wait_idx(j, b8):
        pltpu.make_async_copy(row_hbm.at[base_ch + j], rowc.at[b8],
                              isems[b8]).wait()
        pltpu.make_async_copy(col_hbm.at[base_ch + j], colc.at[b8],
                              isems[b8]).wait()
        pltpu.make_async_copy(w_hbm.at[base_ch + j], wc.at[b8],
                              isems[b8]).wait()

    def start_gather(b4, b8):
        pltpu.async_copy(g_hbm.at[rowc.at[b8, 0]], rin.at[b4], gsems[b4])

    def wait_gather(b4, b8):
        pltpu.make_async_copy(g_hbm.at[rowc.at[b8, 0]], rin.at[b4],
                              gsems[b4]).wait()

    def start_scat(b4, b8):
        pltpu.async_copy(rin.at[b4], accs.at[colc.at[b8, 0]], ssems[b4],
                         add=True)

    def wait_scat(b4, b8):
        pltpu.make_async_copy(rin.at[b4], accs.at[colc.at[b8, 0]],
                              ssems[b4]).wait()

    def scale(b4, b8):
        def one(e2, _):
            for u in range(4):
                e = e2 * 4 + u
                wspl = plsc.load_gather(
                    wc, [jnp.full((16,), b8, jnp.int32),
                         jnp.zeros((16,), jnp.int32),
                         jnp.full((16,), e, jnp.int32)])
                for q in range(8):
                    sl = pl.ds(q * 16, 16)
                    rin[b4, e, sl] = rin[b4, e, sl] * wspl
            return 0
        lax.fori_loop(0, K // 4, one, 0)

    def do_step(j, b4, b8):
        wait_gather(b4, b8)
        scale(b4, b8)
        start_scat(b4, b8)

        @pl.when(j >= 1)
        def _():
            wait_scat((b4 + 2) % 3, (b8 + 5) % 6)

        @pl.when(j + 2 < NCH)
        def _():
            wait_idx(j + 2, (b8 + 2) % 6)
            start_gather((b4 + 2) % 3, (b8 + 2) % 6)

        @pl.when(j + 4 < NCH)
        def _():
            start_idx(j + 4, (b8 + 4) % 6)

    for j0 in range(4):
        start_idx(j0, j0)
    for j0 in range(2):
        wait_idx(j0, j0)
        start_gather(j0, j0)

    def step(i6, _):
        for bb in range(6):
            do_step(i6 * 6 + bb, bb % 3, bb)
        return 0
    NT = NCH // 6
    lax.fori_loop(0, NT, step, 0)
    for t in range(NCH - NT * 6):
        do_step(jnp.int32(NT * 6 + t), t % 3, t)
    wait_scat((NCH - 1) % 3, (NCH - 1) % 6)

    plsc.subcore_barrier()

    def ocp(i, _):
        off = s * RPT + i * ZR
        pltpu.sync_copy(accs.at[pl.ds(off, ZR)], out_hbm.at[c, pl.ds(off, ZR)])
        return 0
    lax.fori_loop(0, RPT // ZR, ocp, 0)


_agg_call = pl.kernel(
    _agg_body,
    out_type=jax.ShapeDtypeStruct((NC, NP_, D), jnp.float32),
    mesh=_mesh,
    compiler_params=_sc_params,
    scratch_types=[
        pltpu.VMEM((NI, 1, K), jnp.int32),
        pltpu.VMEM((NI, 1, K), jnp.int32),
        pltpu.VMEM((NI, 1, K), jnp.float32),
        pltpu.VMEM((NB, K, D), jnp.float32),
        pltpu.VMEM_SHARED((NP_, D), jnp.float32),
        [pltpu.SemaphoreType.DMA] * NI,
        [pltpu.SemaphoreType.DMA] * NB,
        [pltpu.SemaphoreType.DMA] * NB,
    ],
)

BM = 2048
_GRID = NP_ // BM


def _dis_of(degp_ref):
    i = pl.program_id(0)
    deg = jnp.sum(degp_ref[:, 0, pl.ds(i * BM, BM)], axis=0) + 1.0
    return jnp.where(deg > 0, lax.rsqrt(jnp.maximum(deg, 1e-12)), 0.0)


def _tc1_body(x_ref, w_ref, degp_ref, o_ref):
    dis = _dis_of(degp_ref)
    h = jnp.dot(x_ref[...], w_ref[...], preferred_element_type=jnp.float32)
    o_ref[...] = dis[:, None] * h


def _tc2_body(p_ref, g_ref, degp_ref, b_ref, w_ref, o_ref):
    dis = _dis_of(degp_ref)
    s = p_ref[0] + p_ref[1] + g_ref[...]
    out1 = dis[:, None] * s + b_ref[...]
    h = jnp.dot(out1, w_ref[...], preferred_element_type=jnp.float32)
    o_ref[...] = dis[:, None] * h


def _tc3_body(p_ref, g_ref, degp_ref, b_ref, w_ref, bfc_ref, o_ref):
    dis = _dis_of(degp_ref)
    s = p_ref[0] + p_ref[1] + g_ref[...]
    out2 = dis[:, None] * s + b_ref[...]
    y = jnp.dot(out2, w_ref[...], preferred_element_type=jnp.float32)
    y = y + bfc_ref[...]
    o_ref[...] = jnp.round(jnp.clip(y, 0.0, 10.0))


_tc1 = pl.pallas_call(
    _tc1_body,
    grid=(_GRID,),
    in_specs=[
        pl.BlockSpec((BM, D), lambda i: (i, 0)),
        pl.BlockSpec((D, D), lambda i: (0, 0)),
        pl.BlockSpec((NW, 1, NP_), lambda i: (0, 0, 0)),
    ],
    out_specs=pl.BlockSpec((BM, D), lambda i: (i, 0)),
    out_shape=jax.ShapeDtypeStruct((NP_, D), jnp.float32),
)

_tc2 = pl.pallas_call(
    _tc2_body,
    grid=(_GRID,),
    in_specs=[
        pl.BlockSpec((NC, BM, D), lambda i: (0, i, 0)),
        pl.BlockSpec((BM, D), lambda i: (i, 0)),
        pl.BlockSpec((NW, 1, NP_), lambda i: (0, 0, 0)),
        pl.BlockSpec((1, D), lambda i: (0, 0)),
        pl.BlockSpec((D, D), lambda i: (0, 0)),
    ],
    out_specs=pl.BlockSpec((BM, D), lambda i: (i, 0)),
    out_shape=jax.ShapeDtypeStruct((NP_, D), jnp.float32),
)

_tc3 = pl.pallas_call(
    _tc3_body,
    grid=(_GRID,),
    in_specs=[
        pl.BlockSpec((NC, BM, D), lambda i: (0, i, 0)),
        pl.BlockSpec((BM, D), lambda i: (i, 0)),
        pl.BlockSpec((NW, 1, NP_), lambda i: (0, 0, 0)),
        pl.BlockSpec((1, D), lambda i: (0, 0)),
        pl.BlockSpec((D, 1), lambda i: (0, 0)),
        pl.BlockSpec((1, 1), lambda i: (0, 0)),
    ],
    out_specs=pl.BlockSpec((BM, 1), lambda i: (i, 0)),
    out_shape=jax.ShapeDtypeStruct((NP_, 1), jnp.float32),
)


def kernel(x, edge_index, edge_weight, W1, b1, W2, b2, Wfc, bfc):
    row = edge_index[0]
    col = edge_index[1]
    x_p = jnp.pad(x, ((0, NP_ - N), (0, 0)))
    row3 = row.reshape(E // K, 1, K)
    col3 = col.reshape(E // K, 1, K)
    w3 = edge_weight.reshape(E // K, 1, K)
    zrows = jnp.zeros((ZR, D), jnp.float32)
    degp = _deg_call(col, edge_weight)
    g1 = _tc1(x_p, W1, degp)
    p = _agg_call(g1, row3, col3, w3, zrows)
    g2 = _tc2(p, g1, degp, b1.reshape(1, D), W2)
    q = _agg_call(g2, row3, col3, w3, zrows)
    y = _tc3(q, g2, degp, b2.reshape(1, D), Wfc, bfc.reshape(1, 1))
    return y[:N]


# speedup vs baseline: 1.1268x; 1.0197x over previous
import jax
import jax.numpy as jnp
from jax import lax
from jax.experimental import pallas as pl
from jax.experimental.pallas import tpu as pltpu
from jax.experimental.pallas import tpu_sc as plsc

N = 10000
E = 320000
D = 128
NC = 2
NS = 16
NW = NC * NS
EPT = E // NW
K = 80
NCH = EPT // K
NP_ = 10240
RPT = NP_ // NS
ZR = 32

_mesh = plsc.VectorSubcoreMesh(
    core_axis_name="c", subcore_axis_name="s", num_cores=NC, num_subcores=NS)


def _deg_body(col_hbm, w_hbm, out_hbm, colv, wv, degl):
    c = lax.axis_index("c")
    s = lax.axis_index("s")
    wid = c * NS + s
    base = wid * EPT

    def zero(i, _):
        degl[pl.ds(i * 16, 16)] = jnp.zeros((16,), jnp.float32)
        return 0
    lax.fori_loop(0, NP_ // 16, zero, 0)

    pltpu.sync_copy(col_hbm.at[pl.ds(base, EPT)], colv)
    pltpu.sync_copy(w_hbm.at[pl.ds(base, EPT)], wv)

    def scat(i, _):
        c16 = colv[pl.ds(i * 16, 16)]
        w16 = wv[pl.ds(i * 16, 16)]
        plsc.addupdate_scatter(degl, [c16], w16)
        return 0
    lax.fori_loop(0, EPT // 16, scat, 0)

    pltpu.sync_copy(degl, out_hbm.at[wid, 0])


_sc_params = pltpu.CompilerParams(needs_layout_passes=False)

_deg_call = pl.kernel(
    _deg_body,
    out_type=jax.ShapeDtypeStruct((NW, 1, NP_), jnp.float32),
    mesh=_mesh,
    compiler_params=_sc_params,
    scratch_types=[
        pltpu.VMEM((EPT,), jnp.int32),
        pltpu.VMEM((EPT,), jnp.float32),
        pltpu.VMEM((NP_,), jnp.float32),
    ],
)


NB = 3
NI = 6


def _agg_body(g_hbm, row_hbm, col_hbm, w_hbm, z_hbm, out_hbm,
              rowc, colc, wc, rin, accs, isems, gsems, ssems):
    c = lax.axis_index("c")
    s = lax.axis_index("s")
    wid = c * NS + s
    base_ch = wid * NCH

    def zloop(i, _):
        pltpu.sync_copy(z_hbm, accs.at[pl.ds(s * RPT + i * ZR, ZR)])
        return 0
    lax.fori_loop(0, RPT // ZR, zloop, 0)
    plsc.subcore_barrier()

    def start_idx(j, b8):
        pltpu.async_copy(row_hbm.at[base_ch + j], rowc.at[b8], isems[b8])
        pltpu.async_copy(col_hbm.at[base_ch + j], colc.at[b8], isems[b8])
        pltpu.async_copy(w_hbm.at[base_ch + j], wc.at[b8], isems[b8])

    def wait_idx(j, b8):
        pltpu.make_async_copy(row_hbm.at[base_ch + j], rowc.at[b8],
                              isems[b8]).wait()
        pltpu.make_async_copy(col_hbm.at[base_ch + j], colc.at[b8],
                              isems[b8]).wait()
        pltpu.make_async_copy(w_hbm.at[base_ch + j], wc.at[b8],
                              isems[b8]).wait()

    def start_gather(b4, b8):
        pltpu.async_copy(g_hbm.at[rowc.at[b8, 0]], rin.at[b4], gsems[b4])

    def wait_gather(b4, b8):
        pltpu.make_async_copy(g_hbm.at[rowc.at[b8, 0]], rin.at[b4],
                              gsems[b4]).wait()

    def start_scat(b4, b8):
        pltpu.async_copy(rin.at[b4], accs.at[colc.at[b8, 0]], ssems[b4],
                         add=True)

    def wait_scat(b4, b8):
        pltpu.make_async_copy(rin.at[b4], accs.at[colc.at[b8, 0]],
                              ssems[b4]).wait()

    def scale(b4, b8):
        def one(e2, _):
            for u in range(2):
                e = e2 * 2 + u
                wspl = plsc.load_gather(
                    wc, [jnp.full((16,), b8, jnp.int32),
                         jnp.zeros((16,), jnp.int32),
                         jnp.full((16,), e, jnp.int32)])
                for q in range(8):
                    sl = pl.ds(q * 16, 16)
                    rin[b4, e, sl] = rin[b4, e, sl] * wspl
            return 0
        lax.fori_loop(0, K // 2, one, 0)

    def do_step(j, b4, b8):
        wait_gather(b4, b8)
        scale(b4, b8)
        start_scat(b4, b8)

        @pl.when(j >= 1)
        def _():
            wait_scat((b4 + 2) % 3, (b8 + 5) % 6)

        @pl.when(j + 2 < NCH)
        def _():
            wait_idx(j + 2, (b8 + 2) % 6)
            start_gather((b4 + 2) % 3, (b8 + 2) % 6)

        @pl.when(j + 4 < NCH)
        def _():
            start_idx(j + 4, (b8 + 4) % 6)

    for j0 in range(4):
        start_idx(j0, j0)
    for j0 in range(2):
        wait_idx(j0, j0)
        start_gather(j0, j0)

    def step(i6, _):
        for bb in range(6):
            do_step(i6 * 6 + bb, bb % 3, bb)
        return 0
    NT = NCH // 6
    lax.fori_loop(0, NT, step, 0)
    for t in range(NCH - NT * 6):
        do_step(jnp.int32(NT * 6 + t), t % 3, t)
    wait_scat((NCH - 1) % 3, (NCH - 1) % 6)

    plsc.subcore_barrier()

    def ocp(i, _):
        off = s * RPT + i * ZR
        pltpu.sync_copy(accs.at[pl.ds(off, ZR)], out_hbm.at[c, pl.ds(off, ZR)])
        return 0
    lax.fori_loop(0, RPT // ZR, ocp, 0)


_agg_call = pl.kernel(
    _agg_body,
    out_type=jax.ShapeDtypeStruct((NC, NP_, D), jnp.float32),
    mesh=_mesh,
    compiler_params=_sc_params,
    scratch_types=[
        pltpu.VMEM((NI, 1, K), jnp.int32),
        pltpu.VMEM((NI, 1, K), jnp.int32),
        pltpu.VMEM((NI, 1, K), jnp.float32),
        pltpu.VMEM((NB, K, D), jnp.float32),
        pltpu.VMEM_SHARED((NP_, D), jnp.float32),
        [pltpu.SemaphoreType.DMA] * NI,
        [pltpu.SemaphoreType.DMA] * NB,
        [pltpu.SemaphoreType.DMA] * NB,
    ],
)

BM = 2048
_GRID = NP_ // BM


def _dis_of(degp_ref):
    i = pl.program_id(0)
    deg = jnp.sum(degp_ref[:, 0, pl.ds(i * BM, BM)], axis=0) + 1.0
    return jnp.where(deg > 0, lax.rsqrt(jnp.maximum(deg, 1e-12)), 0.0)


def _tc1_body(x_ref, w_ref, degp_ref, o_ref):
    dis = _dis_of(degp_ref)
    h = jnp.dot(x_ref[...], w_ref[...], preferred_element_type=jnp.float32)
    o_ref[...] = dis[:, None] * h


def _tc2_body(p_ref, g_ref, degp_ref, b_ref, w_ref, o_ref):
    dis = _dis_of(degp_ref)
    s = p_ref[0] + p_ref[1] + g_ref[...]
    out1 = dis[:, None] * s + b_ref[...]
    h = jnp.dot(out1, w_ref[...], preferred_element_type=jnp.float32)
    o_ref[...] = dis[:, None] * h


def _tc3_body(p_ref, g_ref, degp_ref, b_ref, w_ref, bfc_ref, o_ref):
    dis = _dis_of(degp_ref)
    s = p_ref[0] + p_ref[1] + g_ref[...]
    out2 = dis[:, None] * s + b_ref[...]
    y = jnp.dot(out2, w_ref[...], preferred_element_type=jnp.float32)
    y = y + bfc_ref[...]
    o_ref[...] = jnp.round(jnp.clip(y, 0.0, 10.0))


_tc1 = pl.pallas_call(
    _tc1_body,
    grid=(_GRID,),
    in_specs=[
        pl.BlockSpec((BM, D), lambda i: (i, 0)),
        pl.BlockSpec((D, D), lambda i: (0, 0)),
        pl.BlockSpec((NW, 1, NP_), lambda i: (0, 0, 0)),
    ],
    out_specs=pl.BlockSpec((BM, D), lambda i: (i, 0)),
    out_shape=jax.ShapeDtypeStruct((NP_, D), jnp.float32),
)

_tc2 = pl.pallas_call(
    _tc2_body,
    grid=(_GRID,),
    in_specs=[
        pl.BlockSpec((NC, BM, D), lambda i: (0, i, 0)),
        pl.BlockSpec((BM, D), lambda i: (i, 0)),
        pl.BlockSpec((NW, 1, NP_), lambda i: (0, 0, 0)),
        pl.BlockSpec((1, D), lambda i: (0, 0)),
        pl.BlockSpec((D, D), lambda i: (0, 0)),
    ],
    out_specs=pl.BlockSpec((BM, D), lambda i: (i, 0)),
    out_shape=jax.ShapeDtypeStruct((NP_, D), jnp.float32),
)

_tc3 = pl.pallas_call(
    _tc3_body,
    grid=(_GRID,),
    in_specs=[
        pl.BlockSpec((NC, BM, D), lambda i: (0, i, 0)),
        pl.BlockSpec((BM, D), lambda i: (i, 0)),
        pl.BlockSpec((NW, 1, NP_), lambda i: (0, 0, 0)),
        pl.BlockSpec((1, D), lambda i: (0, 0)),
        pl.BlockSpec((D, 1), lambda i: (0, 0)),
        pl.BlockSpec((1, 1), lambda i: (0, 0)),
    ],
    out_specs=pl.BlockSpec((BM, 1), lambda i: (i, 0)),
    out_shape=jax.ShapeDtypeStruct((NP_, 1), jnp.float32),
)


def kernel(x, edge_index, edge_weight, W1, b1, W2, b2, Wfc, bfc):
    row = edge_index[0]
    col = edge_index[1]
    x_p = jnp.pad(x, ((0, NP_ - N), (0, 0)))
    row3 = row.reshape(E // K, 1, K)
    col3 = col.reshape(E // K, 1, K)
    w3 = edge_weight.reshape(E // K, 1, K)
    zrows = jnp.zeros((ZR, D), jnp.float32)
    degp = _deg_call(col, edge_weight)
    g1 = _tc1(x_p, W1, degp)
    p = _agg_call(g1, row3, col3, w3, zrows)
    g2 = _tc2(p, g1, degp, b1.reshape(1, D), W2)
    q = _agg_call(g2, row3, col3, w3, zrows)
    y = _tc3(q, g2, degp, b2.reshape(1, D), Wfc, bfc.reshape(1, 1))
    return y[:N]
